# Initial kernel scaffold; baseline (speedup 1.0000x reference)
import jax
import jax.numpy as jnp
from jax.experimental import pallas as pl


def kernel(x, vectors):
    raise NotImplementedError("write your pallas kernel here")


# speedup vs baseline: 4.6579x; 4.6579x over previous
import functools

import jax
import jax.numpy as jnp
from jax import lax
from jax.experimental import pallas as pl
from jax.experimental.pallas import tpu as pltpu
from jax.experimental.pallas import tpu_sc as plsc

NC = 2
NS = 16
NW = NC * NS
GROUP = 128
NBUF = 10


@functools.lru_cache(maxsize=None)
def _build(total_rows: int, vocab: int, embed: int):
    assert total_rows % (NW * GROUP) == 0
    n_groups = total_rows // (NW * GROUP)
    assert n_groups % NBUF == 0
    mesh = plsc.VectorSubcoreMesh(core_axis_name="c", subcore_axis_name="s")

    @functools.partial(
        pl.kernel,
        mesh=mesh,
        compiler_params=pltpu.CompilerParams(use_tc_tiling_on_sc=False),
        out_type=jax.ShapeDtypeStruct((NW, n_groups, GROUP, embed), jnp.float32),
        scratch_types=[
            pltpu.VMEM((n_groups, GROUP), jnp.int32),
            pltpu.VMEM((NBUF, GROUP, embed), jnp.float32),
            pltpu.SemaphoreType.DMA,
            pltpu.SemaphoreType.DMA,
        ],
    )
    def emb_kernel(idx_hbm, table_hbm, out_hbm, idx_v, rows_v, sem_g, sem_o):
        wid = lax.axis_index("s") * NC + lax.axis_index("c")
        pltpu.sync_copy(idx_hbm.at[wid], idx_v)

        @pl.loop(0, n_groups, step=NBUF)
        def _(g0):
            gathers = []
            for b in range(NBUF):
                gathers.append(
                    pltpu.async_copy(
                        table_hbm.at[idx_v.at[g0 + b]], rows_v.at[b], sem_g
                    )
                )
            writes = []
            for b in range(NBUF):
                gathers[b].wait()
                writes.append(
                    pltpu.async_copy(rows_v.at[b], out_hbm.at[wid, g0 + b], sem_o)
                )
            for w in writes:
                w.wait()

    return emb_kernel


def kernel(x, vectors):
    batch, hist = x.shape
    vocab, embed = vectors.shape
    total = batch * hist
    idx = x.astype(jnp.int32).reshape(NW, total // (NW * GROUP), GROUP)
    out = _build(total, vocab, embed)(idx, vectors)
    return out.reshape(batch, hist, embed)
